# baseline (speedup 1.0000x reference)
import jax
import jax.numpy as jnp
from jax.experimental import pallas as pl
from jax.experimental.pallas import tpu as pltpu

_K = 8
_NB = 12
_BF = jnp.bfloat16


def _dot(a, b, dims, prec=jax.lax.Precision.DEFAULT):
    return jax.lax.dot_general(a, b, (dims, ((), ())), precision=prec,
                               preferred_element_type=jnp.float32)


def _split(x):
    hi = x.astype(_BF)
    lo = (x - hi.astype(jnp.float32)).astype(_BF)
    return hi, lo


def _knn_body(xyzr_ref, xyzcb_ref, m_ref, d2_ref):
    B, _, N = xyzr_ref.shape
    inf = jnp.float32(jnp.inf)
    for b in range(B):
        sq_n = jnp.zeros((N, 1), jnp.float32)
        sq_m = jnp.zeros((1, N), jnp.float32)
        for c in range(3):
            xn = xyzcb_ref[b, :, c:c + 1]
            xm = xyzr_ref[b, c:c + 1, :]
            sq_n = sq_n + xn * xn
            sq_m = sq_m + xm * xm
        cross = _dot(xyzcb_ref[b].astype(_BF), xyzr_ref[b].astype(_BF),
                     ((1,), (0,)))
        d2_ref[...] = sq_n + sq_m - 2.0 * cross
        m_ref[b] = jnp.zeros((N, N), _BF)

        def knn_step(k, mval, b=b):
            colids = jax.lax.broadcasted_iota(jnp.int32, (N, N), 1)
            d2 = d2_ref[...]
            am = jnp.min(jnp.where(d2 <= mval, colids, N), axis=1,
                         keepdims=True)
            sel = colids == am
            m_ref[b] = m_ref[b] + sel.astype(_BF)
            d2n = jnp.where(sel, inf, d2)
            d2_ref[...] = d2n
            return jnp.min(d2n, axis=1, keepdims=True)

        jax.lax.fori_loop(0, _K, knn_step,
                          jnp.min(d2_ref[...], axis=1, keepdims=True))


def _main_body(xyzr_ref, x_ref, m_ref, vec_ref, w1h_ref, w2h_ref,
               uce_ref, uco_ref, une_ref, uno_ref, ub_ref,
               newxyz_ref, out_ref, p_ref, s_ref):
    B, _, N = xyzr_ref.shape
    inv_k1 = jnp.float32(1.0 / (_K + 1))

    out_ref[...] = x_ref[...]

    def block(i, carry):
        x = out_ref[...]
        mu = jnp.mean(x, axis=1, keepdims=True)
        xc = x - mu
        var = jnp.mean(xc * xc, axis=1, keepdims=True)
        inv = jax.lax.rsqrt(var + 1e-5)
        vec = vec_ref[i]
        p = jnp.maximum(xc * (inv * vec[:, 0:1]) + vec[:, 1:2], 0.0)
        p_ref[...] = p
        ph = p.astype(_BF)
        for b in range(B):
            s_ref[:, pl.ds(b * N, N)] = _dot(
                ph[:, b * N:(b + 1) * N], m_ref[b], ((1,), (1,)))
        out_ref[...] = out_ref[...] + vec[:, 2:3] + inv_k1 * (
            _dot(w1h_ref[i], ph, ((1,), (0,))))
        sh = s_ref[...].astype(_BF)
        out_ref[...] = out_ref[...] + inv_k1 * (
            _dot(w2h_ref[i], sh, ((1,), (0,))))
        return carry

    jax.lax.fori_loop(0, _NB, block, 0)

    hp = jax.lax.Precision.HIGHEST
    nxe = (_dot(uce_ref[...], p_ref[...], ((1,), (0,)), hp)
           + _dot(une_ref[...], s_ref[...], ((1,), (0,)), hp)) * inv_k1 \
        + ub_ref[:, 0:1]
    nxo = (_dot(uco_ref[...], p_ref[...], ((1,), (0,)), hp)
           + _dot(uno_ref[...], s_ref[...], ((1,), (0,)), hp)) * inv_k1 \
        + ub_ref[:, 1:2]
    for b in range(B):
        xb = xyzr_ref[b]
        newxyz_ref[b, :, 0:N] = nxe[:, b * N:(b + 1) * N] + xb
        newxyz_ref[b, :, N:2 * N] = nxo[:, b * N:(b + 1) * N] + xb


def kernel(xyz, points, bn_gamma, bn_beta, conv1_w, conv1_b, conv2_w, conv2_b,
           up_c_w, up_c_b, up_n_w, up_n_b):
    B, D, N = points.shape
    pts2d = jnp.transpose(points, (1, 0, 2)).reshape(D, B * N)
    xyzcb = jnp.transpose(xyz, (0, 2, 1))
    bias = (conv1_b + float(_K) * conv2_b) / float(_K + 1)
    vec = jnp.stack([bn_gamma, bn_beta, bias], axis=-1)
    ub = jnp.stack([up_c_b[0::2] + float(_K) * up_n_b[0::2],
                    up_c_b[1::2] + float(_K) * up_n_b[1::2]], axis=1) \
        / float(_K + 1)
    w1h = conv1_w.astype(_BF)
    w2h = conv2_w.astype(_BF)

    m = pl.pallas_call(
        _knn_body,
        out_shape=jax.ShapeDtypeStruct((B, N, N), _BF),
        scratch_shapes=[pltpu.VMEM((N, N), jnp.float32)],
    )(xyz, xyzcb)

    newxyz, out2d = pl.pallas_call(
        _main_body,
        out_shape=(jax.ShapeDtypeStruct((B, 3, 2 * N), jnp.float32),
                   jax.ShapeDtypeStruct((D, B * N), jnp.float32)),
        scratch_shapes=[
            pltpu.VMEM((D, B * N), jnp.float32),
            pltpu.VMEM((D, B * N), jnp.float32),
        ],
    )(xyz, pts2d, m, vec, w1h, w2h,
      up_c_w[0::2], up_c_w[1::2], up_n_w[0::2], up_n_w[1::2], ub)
    new_points = out2d.reshape(D, B, N).transpose(1, 0, 2)
    return (newxyz, new_points)


# speedup vs baseline: 1.0159x; 1.0159x over previous
import jax
import jax.numpy as jnp
from jax.experimental import pallas as pl
from jax.experimental.pallas import tpu as pltpu

_K = 8
_NB = 12
_BF = jnp.bfloat16


def _dot(a, b, dims, prec=jax.lax.Precision.DEFAULT):
    return jax.lax.dot_general(a, b, (dims, ((), ())), precision=prec,
                               preferred_element_type=jnp.float32)


def _split(x):
    hi = x.astype(_BF)
    lo = (x - hi.astype(jnp.float32)).astype(_BF)
    return hi, lo


def _knn_body(xyzr_ref, xyzcb_ref, m_ref, d2_ref):
    B, _, N = xyzr_ref.shape
    inf = jnp.float32(jnp.inf)
    for b in range(B):
        sq_n = jnp.zeros((N, 1), jnp.float32)
        sq_m = jnp.zeros((1, N), jnp.float32)
        for c in range(3):
            xn = xyzcb_ref[b, :, c:c + 1]
            xm = xyzr_ref[b, c:c + 1, :]
            sq_n = sq_n + xn * xn
            sq_m = sq_m + xm * xm
        cross = _dot(xyzcb_ref[b].astype(_BF), xyzr_ref[b].astype(_BF),
                     ((1,), (0,)))
        d2_ref[...] = sq_n + sq_m - 2.0 * cross
        m_ref[b] = jnp.zeros((N, N), _BF)

        def knn_step(k, carry, b=b):
            colids = jax.lax.broadcasted_iota(jnp.int32, (N, N), 1)
            d2 = d2_ref[...]
            mval = jnp.min(d2, axis=1, keepdims=True)
            am = jnp.min(jnp.where(d2 <= mval, colids, N), axis=1,
                         keepdims=True)
            sel = colids == am
            m_ref[b] = m_ref[b] + sel.astype(_BF)
            d2_ref[...] = jnp.where(sel, inf, d2)
            return carry

        jax.lax.fori_loop(0, _K, knn_step, 0)


def _main_body(xyzr_ref, x_ref, m_ref, vec_ref, w1h_ref, w2h_ref,
               uce_ref, uco_ref, une_ref, uno_ref, ub_ref,
               newxyz_ref, out_ref, p_ref, s_ref):
    B, _, N = xyzr_ref.shape
    inv_k1 = jnp.float32(1.0 / (_K + 1))

    out_ref[...] = x_ref[...]

    def block(i, carry):
        x = out_ref[...]
        mu = jnp.mean(x, axis=1, keepdims=True)
        xc = x - mu
        var = jnp.mean(xc * xc, axis=1, keepdims=True)
        inv = jax.lax.rsqrt(var + 1e-5)
        vec = vec_ref[i]
        p = jnp.maximum(xc * (inv * vec[:, 0:1]) + vec[:, 1:2], 0.0)
        p_ref[...] = p
        ph = p.astype(_BF)
        for b in range(B):
            s_ref[:, pl.ds(b * N, N)] = _dot(
                ph[:, b * N:(b + 1) * N], m_ref[b], ((1,), (1,)))
        out_ref[...] = out_ref[...] + vec[:, 2:3] + inv_k1 * (
            _dot(w1h_ref[i], ph, ((1,), (0,))))
        sh = s_ref[...].astype(_BF)
        out_ref[...] = out_ref[...] + inv_k1 * (
            _dot(w2h_ref[i], sh, ((1,), (0,))))
        return carry

    jax.lax.fori_loop(0, _NB, block, 0)

    hp = jax.lax.Precision.HIGHEST
    nxe = (_dot(uce_ref[...], p_ref[...], ((1,), (0,)), hp)
           + _dot(une_ref[...], s_ref[...], ((1,), (0,)), hp)) * inv_k1 \
        + ub_ref[:, 0:1]
    nxo = (_dot(uco_ref[...], p_ref[...], ((1,), (0,)), hp)
           + _dot(uno_ref[...], s_ref[...], ((1,), (0,)), hp)) * inv_k1 \
        + ub_ref[:, 1:2]
    for b in range(B):
        xb = xyzr_ref[b]
        newxyz_ref[b, :, 0:N] = nxe[:, b * N:(b + 1) * N] + xb
        newxyz_ref[b, :, N:2 * N] = nxo[:, b * N:(b + 1) * N] + xb


def kernel(xyz, points, bn_gamma, bn_beta, conv1_w, conv1_b, conv2_w, conv2_b,
           up_c_w, up_c_b, up_n_w, up_n_b):
    B, D, N = points.shape
    pts2d = jnp.transpose(points, (1, 0, 2)).reshape(D, B * N)
    xyzcb = jnp.transpose(xyz, (0, 2, 1))
    bias = (conv1_b + float(_K) * conv2_b) / float(_K + 1)
    vec = jnp.stack([bn_gamma, bn_beta, bias], axis=-1)
    ub = jnp.stack([up_c_b[0::2] + float(_K) * up_n_b[0::2],
                    up_c_b[1::2] + float(_K) * up_n_b[1::2]], axis=1) \
        / float(_K + 1)
    w1h = conv1_w.astype(_BF)
    w2h = conv2_w.astype(_BF)

    m = pl.pallas_call(
        _knn_body,
        out_shape=jax.ShapeDtypeStruct((B, N, N), _BF),
        scratch_shapes=[pltpu.VMEM((N, N), jnp.float32)],
    )(xyz, xyzcb)

    newxyz, out2d = pl.pallas_call(
        _main_body,
        out_shape=(jax.ShapeDtypeStruct((B, 3, 2 * N), jnp.float32),
                   jax.ShapeDtypeStruct((D, B * N), jnp.float32)),
        scratch_shapes=[
            pltpu.VMEM((D, B * N), jnp.float32),
            pltpu.VMEM((D, B * N), jnp.float32),
        ],
    )(xyz, pts2d, m, vec, w1h, w2h,
      up_c_w[0::2], up_c_w[1::2], up_n_w[0::2], up_n_w[1::2], ub)
    new_points = out2d.reshape(D, B, N).transpose(1, 0, 2)
    return (newxyz, new_points)


# speedup vs baseline: 1.0161x; 1.0003x over previous
import jax
import jax.numpy as jnp
from jax.experimental import pallas as pl
from jax.experimental.pallas import tpu as pltpu

_K = 8
_NB = 12
_BF = jnp.bfloat16


def _dot(a, b, dims, prec=jax.lax.Precision.DEFAULT):
    return jax.lax.dot_general(a, b, (dims, ((), ())), precision=prec,
                               preferred_element_type=jnp.float32)


def _knn_body(xyzr_ref, xyzcb_ref, m_ref, d2_ref):
    B, _, N = xyzr_ref.shape
    inf = jnp.float32(jnp.inf)
    for b in range(B):
        sq_n = jnp.zeros((N, 1), jnp.float32)
        sq_m = jnp.zeros((1, N), jnp.float32)
        for c in range(3):
            xn = xyzcb_ref[b, :, c:c + 1]
            xm = xyzr_ref[b, c:c + 1, :]
            sq_n = sq_n + xn * xn
            sq_m = sq_m + xm * xm
        cross = _dot(xyzcb_ref[b].astype(_BF), xyzr_ref[b].astype(_BF),
                     ((1,), (0,)))
        d2_ref[...] = sq_n + sq_m - 2.0 * cross
        m_ref[b] = jnp.zeros((N, N), _BF)

        def knn_step(k, carry, b=b):
            colids = jax.lax.broadcasted_iota(jnp.int32, (N, N), 1)
            d2 = d2_ref[...]
            mval = jnp.min(d2, axis=1, keepdims=True)
            am = jnp.min(jnp.where(d2 <= mval, colids, N), axis=1,
                         keepdims=True)
            sel = colids == am
            m_ref[b] = m_ref[b] + sel.astype(_BF)
            d2_ref[...] = jnp.where(sel, inf, d2)
            return carry

        jax.lax.fori_loop(0, _K, knn_step, 0)


def _main_body(xyzr_ref, x_ref, m_ref, vec_ref, w1h_ref, w2h_ref,
               uce_ref, uco_ref, une_ref, uno_ref, ub_ref,
               newxyz_ref, out_ref, p_ref, s_ref):
    B, _, N = xyzr_ref.shape
    inv_k1 = jnp.float32(1.0 / (_K + 1))

    out_ref[...] = x_ref[...]

    def block(i, carry):
        x = out_ref[...]
        mu = jnp.mean(x, axis=1, keepdims=True)
        xc = x - mu
        var = jnp.mean(xc * xc, axis=1, keepdims=True)
        inv = jax.lax.rsqrt(var + 1e-5)
        vec = vec_ref[i]
        p = jnp.maximum(xc * (inv * vec[:, 0:1]) + vec[:, 1:2], 0.0)
        p_ref[...] = p
        ph = p.astype(_BF)
        for b in range(B):
            s_ref[:, pl.ds(b * N, N)] = _dot(
                ph[:, b * N:(b + 1) * N], m_ref[b], ((1,), (1,)))
        out_ref[...] = out_ref[...] + vec[:, 2:3] + inv_k1 * (
            _dot(w1h_ref[i], ph, ((1,), (0,))))
        sh = s_ref[...].astype(_BF)
        out_ref[...] = out_ref[...] + inv_k1 * (
            _dot(w2h_ref[i], sh, ((1,), (0,))))
        return carry

    jax.lax.fori_loop(0, _NB, block, 0)

    hp = jax.lax.Precision.HIGHEST
    nxe = (_dot(uce_ref[...], p_ref[...], ((1,), (0,)), hp)
           + _dot(une_ref[...], s_ref[...], ((1,), (0,)), hp)) * inv_k1 \
        + ub_ref[:, 0:1]
    nxo = (_dot(uco_ref[...], p_ref[...], ((1,), (0,)), hp)
           + _dot(uno_ref[...], s_ref[...], ((1,), (0,)), hp)) * inv_k1 \
        + ub_ref[:, 1:2]
    for b in range(B):
        xb = xyzr_ref[b]
        newxyz_ref[b, :, 0:N] = nxe[:, b * N:(b + 1) * N] + xb
        newxyz_ref[b, :, N:2 * N] = nxo[:, b * N:(b + 1) * N] + xb


def kernel(xyz, points, bn_gamma, bn_beta, conv1_w, conv1_b, conv2_w, conv2_b,
           up_c_w, up_c_b, up_n_w, up_n_b):
    B, D, N = points.shape
    pts2d = jnp.transpose(points, (1, 0, 2)).reshape(D, B * N)
    xyzcb = jnp.transpose(xyz, (0, 2, 1))
    bias = (conv1_b + float(_K) * conv2_b) / float(_K + 1)
    vec = jnp.stack([bn_gamma, bn_beta, bias], axis=-1)
    ub = jnp.stack([up_c_b[0::2] + float(_K) * up_n_b[0::2],
                    up_c_b[1::2] + float(_K) * up_n_b[1::2]], axis=1) \
        / float(_K + 1)
    w1h = conv1_w.astype(_BF)
    w2h = conv2_w.astype(_BF)

    m = pl.pallas_call(
        _knn_body,
        out_shape=jax.ShapeDtypeStruct((B, N, N), _BF),
        scratch_shapes=[pltpu.VMEM((N, N), jnp.float32)],
    )(xyz, xyzcb)

    newxyz, out2d = pl.pallas_call(
        _main_body,
        out_shape=(jax.ShapeDtypeStruct((B, 3, 2 * N), jnp.float32),
                   jax.ShapeDtypeStruct((D, B * N), jnp.float32)),
        scratch_shapes=[
            pltpu.VMEM((D, B * N), jnp.float32),
            pltpu.VMEM((D, B * N), jnp.float32),
        ],
    )(xyz, pts2d, m, vec, w1h, w2h,
      up_c_w[0::2], up_c_w[1::2], up_n_w[0::2], up_n_w[1::2], ub)
    new_points = out2d.reshape(D, B, N).transpose(1, 0, 2)
    return (newxyz, new_points)
